# baseline (speedup 1.0000x reference)
import functools

import numpy as np
import jax
import jax.numpy as jnp
from jax import lax
from jax.experimental import pallas as pl
from jax.experimental.pallas import tpu as pltpu
from jax.experimental.pallas import tpu_sc as plsc

_MODEL_DIM = 1024
_PAD = 0
_BATCH = 4
_SEQ = 2048

_N = _BATCH * _SEQ
_NC, _NS, _L = 2, 16, 16
_NW = _NC * _NS
_P = _SEQ // _NW
_C = 16
_NH = _P // _C
_NT = _NH * _BATCH
_VPR = _MODEL_DIM // _L
_NB = 4
_NS1 = _SEQ // _C


def _pe_factors():
    d = _MODEL_DIM
    div = np.exp(np.arange(0, d, 2, dtype=np.float32)
                 * (-np.log(10000.0) / d))

    def rows(pos):
        ang = pos[:, None] * div[None, :]
        return np.sin(ang), np.cos(ang)

    sa, ca = rows(np.arange(_NS1, dtype=np.float32) * _C)
    xc = np.zeros((_NS1, 2, d), dtype=np.float32)
    xc[:, 0, 0::2] = sa
    xc[:, 0, 1::2] = ca
    xc[:, 1, 0::2] = ca
    xc[:, 1, 1::2] = -sa

    sb, cb = rows(np.arange(_C, dtype=np.float32))
    yc = np.zeros((_C, 2, d), dtype=np.float32)
    yc[:, 0, 0::2] = cb
    yc[:, 0, 1::2] = cb
    yc[:, 1, 0::2] = sb
    yc[:, 1, 1::2] = sb
    return xc.reshape(-1), yc.reshape(-1)


_mesh = plsc.VectorSubcoreMesh(core_axis_name="c", subcore_axis_name="s")

_scratch = (
    [pltpu.VMEM((_BATCH * _P,), jnp.int32)]
    + [pltpu.VMEM((_C, _MODEL_DIM), jnp.float32) for _ in range(_NB)]
    + [pltpu.VMEM((_C * _MODEL_DIM,), jnp.float32)]
    + [pltpu.VMEM((2 * _MODEL_DIM,), jnp.float32) for _ in range(2)]
    + [pltpu.VMEM((_C * 2 * _MODEL_DIM,), jnp.float32)]
    + [pltpu.SemaphoreType.DMA for _ in range(_NB + _NB + 2 + 1)]
)


@functools.partial(
    pl.kernel,
    mesh=_mesh,
    out_type=jax.ShapeDtypeStruct((_N, _MODEL_DIM), jnp.float32),
    scratch_types=_scratch,
)
def _emb_body(tok_hbm, table_hbm, xc_hbm, yc_hbm, out_hbm, idx_v, *bufs):
    rows_v = bufs[:_NB]
    pe_v = bufs[_NB]
    xc_v = bufs[_NB + 1:_NB + 3]
    yc_v = bufs[_NB + 3]
    sems = bufs[_NB + 4:]
    semg = sems[:_NB]
    semo = sems[_NB:2 * _NB]
    semx = sems[2 * _NB:2 * _NB + 2]
    semy = sems[2 * _NB + 2]

    w = lax.axis_index("s") * _NC + lax.axis_index("c")
    p0 = w * _P
    d = _MODEL_DIM

    def tok_off(h, b):
        return b * _P + h * _C

    def out_row0(h, b):
        return b * _SEQ + p0 + h * _C

    def issue_gather(t):
        h, b = divmod(t, _BATCH)
        idx = idx_v.at[pl.ds(tok_off(h, b), _C)]
        return pltpu.async_copy(table_hbm.at[idx], rows_v[t % _NB],
                                semg[t % _NB])

    def issue_xc(h):
        src = xc_hbm.at[pl.ds((w * _NH + h) * 2 * d, 2 * d)]
        return pltpu.async_copy(src, xc_v[h % 2], semx[h % 2])

    def issue_out(t):
        h, b = divmod(t, _BATCH)
        return pltpu.async_copy(rows_v[t % _NB],
                                out_hbm.at[pl.ds(out_row0(h, b), _C)],
                                semo[t % _NB])

    def build_pe(h):
        xc = xc_v[h % 2]

        def vloop(v, carry):
            x0 = xc[pl.ds(v * _L, _L)]
            x1 = xc[pl.ds(d + v * _L, _L)]

            @plsc.parallel_loop(0, _C, unroll=4)
            def _(j):
                y0 = yc_v[pl.ds(j * 2 * d + v * _L, _L)]
                y1 = yc_v[pl.ds(j * 2 * d + d + v * _L, _L)]
                pe_v[pl.ds(j * d + v * _L, _L)] = x0 * y0 + x1 * y1

            return carry

        lax.fori_loop(0, _VPR, vloop, 0)

    def compute(t):
        h, b = divmod(t, _BATCH)
        rows = rows_v[t % _NB]
        tokv = idx_v[pl.ds(tok_off(h, b), _C)]

        def row(j, carry):
            tok_b = tokv.at[jnp.full((_L,), 0, jnp.int32) + j].get(
                mode="promise_in_bounds")
            m = jnp.where(tok_b != _PAD, jnp.float32(1.0), jnp.float32(0.0))

            @plsc.parallel_loop(0, _VPR, unroll=8)
            def _(v):
                sl = (j, pl.ds(v * _L, _L))
                rows[sl] = rows[sl] * m + pe_v[pl.ds(j * d + v * _L, _L)]

            return carry

        lax.fori_loop(0, _C, row, 0)

    for b in range(_BATCH):
        pltpu.sync_copy(tok_hbm.at[b, pl.ds(p0, _P)],
                        idx_v.at[pl.ds(b * _P, _P)])
    yc_desc = pltpu.async_copy(yc_hbm, yc_v, semy)
    xc_desc = {0: issue_xc(0), 1: issue_xc(1)}
    g_desc = {t: issue_gather(t) for t in range(_NB)}
    o_desc = {}

    for t in range(_NT):
        h, b = divmod(t, _BATCH)
        if t >= 1 and t + _NB - 1 < _NT:
            o_desc.pop(t - 1).wait()
            g_desc[t + _NB - 1] = issue_gather(t + _NB - 1)
        if b == 0:
            if h == 0:
                yc_desc.wait()
            xc_desc.pop(h).wait()
            if h >= 1 and h + 1 < _NH:
                xc_desc[h + 1] = issue_xc(h + 1)
            build_pe(h)
        g_desc.pop(t).wait()
        compute(t)
        o_desc[t] = issue_out(t)

    for t in sorted(o_desc):
        o_desc.pop(t).wait()


def kernel(tokens, table):
    xc, yc = _pe_factors()
    out = _emb_body(tokens, table, jnp.asarray(xc), jnp.asarray(yc))
    return out.reshape(_BATCH, _SEQ, _MODEL_DIM)


# speedup vs baseline: 1.2669x; 1.2669x over previous
import functools

import numpy as np
import jax
import jax.numpy as jnp
from jax import lax
from jax.experimental import pallas as pl
from jax.experimental.pallas import tpu as pltpu
from jax.experimental.pallas import tpu_sc as plsc

_MODEL_DIM = 1024
_PAD = 0
_BATCH = 4
_SEQ = 2048

_N = _BATCH * _SEQ
_NC, _NS, _L = 2, 16, 16
_NW = _NC * _NS
_P = _SEQ // _NW
_C = 16
_NH = _P // _C
_NT = _NH * _BATCH
_VPR = _MODEL_DIM // _L
_NB = 5


def _sinusoidal_pe(max_len, d):
    pos = np.arange(max_len, dtype=np.float32)[:, None]
    div = np.exp(np.arange(0, d, 2, dtype=np.float32) * (-np.log(10000.0) / d))
    pe = np.zeros((max_len, d), dtype=np.float32)
    pe[:, 0::2] = np.sin(pos * div)
    pe[:, 1::2] = np.cos(pos * div)
    return pe


_mesh = plsc.VectorSubcoreMesh(core_axis_name="c", subcore_axis_name="s")

_scratch = (
    [pltpu.VMEM((_BATCH * _P,), jnp.int32)]
    + [pltpu.VMEM((_C, _MODEL_DIM), jnp.float32) for _ in range(_NB)]
    + [pltpu.VMEM((_C * _MODEL_DIM,), jnp.float32) for _ in range(2)]
    + [pltpu.SemaphoreType.DMA for _ in range(_NB + _NB + 2)]
)


@functools.partial(
    pl.kernel,
    mesh=_mesh,
    out_type=jax.ShapeDtypeStruct((_N, _MODEL_DIM), jnp.float32),
    scratch_types=_scratch,
)
def _emb_body(tok_hbm, table_hbm, pe_hbm, out_hbm, idx_v, *bufs):
    rows_v = bufs[:_NB]
    pe_v = bufs[_NB:_NB + 2]
    semg = bufs[_NB + 2:_NB + 2 + _NB]
    semo = bufs[_NB + 2 + _NB:_NB + 2 + 2 * _NB]
    semp = bufs[_NB + 2 + 2 * _NB:]

    w = lax.axis_index("s") * _NC + lax.axis_index("c")
    p0 = w * _P

    def tok_off(h, b):
        return b * _P + h * _C

    def out_row0(h, b):
        return b * _SEQ + p0 + h * _C

    def issue_gather(t):
        h, b = divmod(t, _BATCH)
        idx = idx_v.at[pl.ds(tok_off(h, b), _C)]
        return pltpu.async_copy(table_hbm.at[idx], rows_v[t % _NB],
                                semg[t % _NB])

    def issue_pe(h):
        src = pe_hbm.at[pl.ds((p0 + h * _C) * _MODEL_DIM, _C * _MODEL_DIM)]
        return pltpu.async_copy(src, pe_v[h % 2], semp[h % 2])

    def issue_out(t):
        h, b = divmod(t, _BATCH)
        return pltpu.async_copy(rows_v[t % _NB],
                                out_hbm.at[pl.ds(out_row0(h, b), _C)],
                                semo[t % _NB])

    def bcast_mask(tokv, j):
        tok_b = tokv.at[jnp.full((_L,), 0, jnp.int32) + j].get(
            mode="promise_in_bounds")
        return jnp.where(tok_b != _PAD, jnp.float32(1.0), jnp.float32(0.0))

    def compute_pair(p):
        h = p // 2
        b0 = 2 * (p % 2)
        pe = pe_v[h % 2]
        r0 = rows_v[(2 * p) % _NB]
        r1 = rows_v[(2 * p + 1) % _NB]
        tok0 = idx_v[pl.ds(tok_off(h, b0), _C)]
        tok1 = idx_v[pl.ds(tok_off(h, b0 + 1), _C)]

        def row(j, carry):
            m0 = bcast_mask(tok0, j)
            m1 = bcast_mask(tok1, j)

            @plsc.parallel_loop(0, _VPR, unroll=4)
            def _(v):
                pv = pe[pl.ds(j * _MODEL_DIM + v * _L, _L)]
                sl = (j, pl.ds(v * _L, _L))
                r0[sl] = r0[sl] * m0 + pv
                r1[sl] = r1[sl] * m1 + pv

            return carry

        lax.fori_loop(0, _C, row, 0)

    idx_desc = [
        pltpu.async_copy(tok_hbm.at[b, pl.ds(p0, _P)],
                         idx_v.at[pl.ds(b * _P, _P)], semg[b])
        for b in range(_BATCH)
    ]
    for desc in idx_desc:
        desc.wait()
    pe_desc = {0: issue_pe(0), 1: issue_pe(1)}
    g_desc = {t: issue_gather(t) for t in range(_NB)}
    o_desc = {}

    for p in range(_NT // 2):
        t0, t1 = 2 * p, 2 * p + 1
        h = p // 2
        if t0 >= 1 and t0 + _NB - 1 < _NT:
            o_desc.pop(t0 - 1).wait()
            g_desc[t0 + _NB - 1] = issue_gather(t0 + _NB - 1)
        if p % 2 == 0:
            pe_desc.pop(h).wait()
            if h >= 1 and h + 1 < _NH:
                pe_desc[h + 1] = issue_pe(h + 1)
        g_desc.pop(t0).wait()
        g_desc.pop(t1).wait()
        compute_pair(p)
        o_desc[t0] = issue_out(t0)
        o_desc[t1] = issue_out(t1)
        if t1 + _NB - 1 < _NT:
            o_desc.pop(t1 - 1).wait()
            g_desc[t1 + _NB - 1] = issue_gather(t1 + _NB - 1)

    for t in sorted(o_desc):
        o_desc.pop(t).wait()


def kernel(tokens, table):
    pe = jnp.asarray(_sinusoidal_pe(_SEQ, _MODEL_DIM).reshape(-1))
    out = _emb_body(tokens, table, pe)
    return out.reshape(_BATCH, _SEQ, _MODEL_DIM)
